# Initial kernel scaffold; baseline (speedup 1.0000x reference)
import jax
import jax.numpy as jnp
from jax.experimental import pallas as pl


def kernel(x, hyperedge_index, weight, att):
    raise NotImplementedError("write your pallas kernel here")


# speedup vs baseline: 5.2591x; 5.2591x over previous
import jax
import jax.numpy as jnp
from jax.experimental import pallas as pl
from jax.experimental.pallas import tpu as pltpu

N = 10000
E = 160000
H = 128
C = 128
CHUNK = 4000
NBLK = E // CHUNK


def _scatter_kernel(idx_ref, a_ref):
    @pl.when(pl.program_id(0) == 0)
    def _():
        a_ref[...] = jnp.zeros_like(a_ref)

    iota_h = jax.lax.broadcasted_iota(jnp.int32, (1, H), 1)

    def body(i, carry):
        n = idx_ref[0, 0, i]
        h = idx_ref[0, 1, i]
        add = (iota_h == h).astype(jnp.float32)
        a_ref[pl.ds(n, 1), :] += add
        return carry

    jax.lax.fori_loop(0, CHUNK, body, 0)


def _dense_kernel(a_ref, x_ref, w_ref, att_ref, out_ref, cons_ref):
    A = a_ref[...]
    xw = jnp.dot(x_ref[...], w_ref[...],
                 preferred_element_type=jnp.float32)
    q = jax.lax.dot_general(A, xw, (((0,), (0,)), ((), ())),
                            preferred_element_type=jnp.float32,
                 precision=jax.lax.Precision.HIGHEST)
    dn = jnp.sum(A, axis=1, keepdims=True)
    dege = jnp.sum(A, axis=0, keepdims=True)

    att1 = att_ref[:, :C]
    att2 = att_ref[:, C:]
    s_n = jax.lax.dot_general(xw, att1, (((1,), (1,)), ((), ())),
                              preferred_element_type=jnp.float32,
                 precision=jax.lax.Precision.HIGHEST)
    s_h = jax.lax.dot_general(att2, q, (((1,), (1,)), ((), ())),
                              preferred_element_type=jnp.float32,
                 precision=jax.lax.Precision.HIGHEST)

    aa = s_n + s_h
    aa = jnp.where(aa >= 0, aa, 0.2 * aa)
    mask = A > 0
    amax = jnp.max(jnp.where(mask, aa, -jnp.inf), axis=1, keepdims=True)
    amax = jnp.where(jnp.isfinite(amax), amax, 0.0)
    ex = jnp.where(mask, jnp.exp(aa - amax), 0.0)
    denom = jnp.sum(A * ex, axis=1, keepdims=True)
    Wm = A * ex / (denom + 1e-16)

    Bn = jnp.where(dege > 0, 1.0 / jnp.maximum(dege, 1.0), 0.0)
    out_e = jnp.transpose(Bn) * jax.lax.dot_general(
        Wm, xw, (((0,), (0,)), ((), ())),
        preferred_element_type=jnp.float32,
                 precision=jax.lax.Precision.HIGHEST)
    out_n = dn * jnp.dot(Wm, out_e,
                         preferred_element_type=jnp.float32,
                 precision=jax.lax.Precision.HIGHEST)
    out_ref[...] = out_n

    qrow = jnp.sum(q, axis=1, keepdims=True)
    meanval = (jnp.sum(q) - jnp.dot(dege, qrow,
                                    preferred_element_type=jnp.float32,
                 precision=jax.lax.Precision.HIGHEST)[0, 0]) \
        / float(E * C)
    inner = jnp.dot(q, jnp.transpose(q),
                    preferred_element_type=jnp.float32,
                 precision=jax.lax.Precision.HIGHEST)
    sq = jnp.sum(q * q, axis=1, keepdims=True)
    nrm = jnp.sqrt(sq + 1e-12)
    alpha_km = inner / (nrm * jnp.transpose(nrm))
    dist = jnp.sqrt(jnp.maximum(sq + jnp.transpose(sq) - 2.0 * inner, 1e-12))
    li = alpha_km * dist + (1.0 - alpha_km) * jnp.maximum(4.2 - dist, 0.0)
    loss_hyper = jnp.sum(jnp.abs(li)) / float((H + 1) ** 2)
    cons_ref[...] = jnp.full((1, 1), jnp.abs(meanval) + loss_hyper,
                             dtype=jnp.float32)


def kernel(x, hyperedge_index, weight, att):
    x2 = x[0]
    idx3 = jnp.transpose(hyperedge_index.reshape(2, NBLK, CHUNK), (1, 0, 2))

    A = pl.pallas_call(
        _scatter_kernel,
        grid=(NBLK,),
        in_specs=[pl.BlockSpec((1, 2, CHUNK), lambda i: (i, 0, 0),
                               memory_space=pltpu.SMEM)],
        out_specs=pl.BlockSpec((N, H), lambda i: (0, 0)),
        out_shape=jax.ShapeDtypeStruct((N, H), jnp.float32),
    )(idx3)

    att_flat = att.reshape(1, 2 * C)
    out_n, cons = pl.pallas_call(
        _dense_kernel,
        out_shape=(jax.ShapeDtypeStruct((N, C), jnp.float32),
                   jax.ShapeDtypeStruct((1, 1), jnp.float32)),
    )(A, x2, weight, att_flat)

    return out_n[None], cons[0, 0]


# speedup vs baseline: 10.3872x; 1.9751x over previous
import jax
import jax.numpy as jnp
from jax.experimental import pallas as pl
from jax.experimental.pallas import tpu as pltpu

N = 10000
E = 160000
H = 128
C = 128
CHUNK = 4000
NBLK = E // CHUNK


def _scatter_kernel(idx_ref, a_ref):
    @pl.when(pl.program_id(0) == 0)
    def _():
        a_ref[...] = jnp.zeros_like(a_ref)

    iota_h = jax.lax.broadcasted_iota(jnp.int32, (1, H), 1)

    def body(i, carry):
        n = idx_ref[0, 0, i]
        h = idx_ref[0, 1, i]
        add = (iota_h == h).astype(jnp.float32)
        a_ref[pl.ds(n, 1), :] += add
        return carry

    jax.lax.fori_loop(0, CHUNK, body, 0, unroll=8)


def _dense_kernel(a_ref, x_ref, w_ref, att_ref, out_ref, cons_ref):
    A = a_ref[...]
    xw = jnp.dot(x_ref[...], w_ref[...],
                 preferred_element_type=jnp.float32)
    q = jax.lax.dot_general(A, xw, (((0,), (0,)), ((), ())),
                            preferred_element_type=jnp.float32,
                 precision=jax.lax.Precision.HIGHEST)
    dn = jnp.sum(A, axis=1, keepdims=True)
    dege = jnp.sum(A, axis=0, keepdims=True)

    att1 = att_ref[:, :C]
    att2 = att_ref[:, C:]
    s_n = jax.lax.dot_general(xw, att1, (((1,), (1,)), ((), ())),
                              preferred_element_type=jnp.float32,
                 precision=jax.lax.Precision.HIGHEST)
    s_h = jax.lax.dot_general(att2, q, (((1,), (1,)), ((), ())),
                              preferred_element_type=jnp.float32,
                 precision=jax.lax.Precision.HIGHEST)

    aa = s_n + s_h
    aa = jnp.where(aa >= 0, aa, 0.2 * aa)
    mask = A > 0
    amax = jnp.max(jnp.where(mask, aa, -jnp.inf), axis=1, keepdims=True)
    amax = jnp.where(jnp.isfinite(amax), amax, 0.0)
    ex = jnp.where(mask, jnp.exp(aa - amax), 0.0)
    denom = jnp.sum(A * ex, axis=1, keepdims=True)
    Wm = A * ex / (denom + 1e-16)

    Bn = jnp.where(dege > 0, 1.0 / jnp.maximum(dege, 1.0), 0.0)
    out_e = jnp.transpose(Bn) * jax.lax.dot_general(
        Wm, xw, (((0,), (0,)), ((), ())),
        preferred_element_type=jnp.float32,
                 precision=jax.lax.Precision.HIGHEST)
    out_n = dn * jnp.dot(Wm, out_e,
                         preferred_element_type=jnp.float32,
                 precision=jax.lax.Precision.HIGHEST)
    out_ref[...] = out_n

    qrow = jnp.sum(q, axis=1, keepdims=True)
    meanval = (jnp.sum(q) - jnp.dot(dege, qrow,
                                    preferred_element_type=jnp.float32,
                 precision=jax.lax.Precision.HIGHEST)[0, 0]) \
        / float(E * C)
    inner = jnp.dot(q, jnp.transpose(q),
                    preferred_element_type=jnp.float32,
                 precision=jax.lax.Precision.HIGHEST)
    sq = jnp.sum(q * q, axis=1, keepdims=True)
    nrm = jnp.sqrt(sq + 1e-12)
    alpha_km = inner / (nrm * jnp.transpose(nrm))
    dist = jnp.sqrt(jnp.maximum(sq + jnp.transpose(sq) - 2.0 * inner, 1e-12))
    li = alpha_km * dist + (1.0 - alpha_km) * jnp.maximum(4.2 - dist, 0.0)
    loss_hyper = jnp.sum(jnp.abs(li)) / float((H + 1) ** 2)
    cons_ref[...] = jnp.full((1, 1), jnp.abs(meanval) + loss_hyper,
                             dtype=jnp.float32)


def kernel(x, hyperedge_index, weight, att):
    x2 = x[0]
    idx3 = jnp.transpose(hyperedge_index.reshape(2, NBLK, CHUNK), (1, 0, 2))

    A = pl.pallas_call(
        _scatter_kernel,
        grid=(NBLK,),
        in_specs=[pl.BlockSpec((1, 2, CHUNK), lambda i: (i, 0, 0),
                               memory_space=pltpu.SMEM)],
        out_specs=pl.BlockSpec((N, H), lambda i: (0, 0)),
        out_shape=jax.ShapeDtypeStruct((N, H), jnp.float32),
    )(idx3)

    att_flat = att.reshape(1, 2 * C)
    out_n, cons = pl.pallas_call(
        _dense_kernel,
        out_shape=(jax.ShapeDtypeStruct((N, C), jnp.float32),
                   jax.ShapeDtypeStruct((1, 1), jnp.float32)),
    )(A, x2, weight, att_flat)

    return out_n[None], cons[0, 0]
